# baseline (speedup 1.0000x reference)
import jax
import jax.numpy as jnp
from jax import lax
from jax.experimental import pallas as pl
from jax.experimental.pallas import tpu as pltpu
from jax.experimental.pallas import tpu_sc as plsc

TOKENS = 32768
ACTORS = 32768
DM = 2048
NCH = 16

ROW_BLK = 2048
NWORKERS = 32
BPW = ACTORS // NWORKERS

SC_TOKENS = 4096
TC_TOKENS = TOKENS - SC_TOKENS
TPW = SC_TOKENS // NWORKERS
TB = 8
KC = 16
KN = DM // KC


def _proj_body(x_ref, w_ref, b_ref, lg_ref):
    logits = jnp.dot(x_ref[...], w_ref[...], preferred_element_type=jnp.float32)
    lg_ref[...] = logits + b_ref[...]


def _project(x, W, b2):
    return pl.pallas_call(
        _proj_body,
        grid=(TC_TOKENS // ROW_BLK,),
        in_specs=[
            pl.BlockSpec((ROW_BLK, DM), lambda i: (i, 0)),
            pl.BlockSpec((DM, NCH), lambda i: (0, 0)),
            pl.BlockSpec((1, NCH), lambda i: (0, 0)),
        ],
        out_specs=pl.BlockSpec((ROW_BLK, NCH), lambda i: (i, 0)),
        out_shape=jax.ShapeDtypeStruct((TC_TOKENS, NCH), jnp.float32),
        compiler_params=pltpu.CompilerParams(
            vmem_limit_bytes=100 * 1024 * 1024),
    )(x[:TC_TOKENS], W, b2)


def _scproj_body(x_hbm, w_hbm, b_hbm, out_hbm, w_v, b_v, xb_v, lg_v):
    ncores = lax.axis_size("c")
    wid = lax.axis_index("s") * ncores + lax.axis_index("c")
    t0 = TC_TOKENS + wid * TPW
    o0 = wid * TPW
    pltpu.sync_copy(w_hbm, w_v)
    pltpu.sync_copy(b_hbm, b_v)
    bvec = b_v[...]

    def block(bi, carry):
        pltpu.sync_copy(x_hbm.at[pl.ds(t0 + bi * TB, TB)], xb_v)

        def kchunk(kc, accs):
            k0 = kc * KC
            wvs = [w_v[k0 + kk, :] for kk in range(KC)]
            new = list(accs)
            for t in range(TB):
                xv = xb_v[t, pl.ds(k0, KC)]
                for kk in range(KC):
                    new[t] = new[t] + xv[kk] * wvs[kk]
            return tuple(new)

        accs = lax.fori_loop(0, KN, kchunk, (bvec,) * TB)
        for t in range(TB):
            lg_v[t, :] = accs[t]
        pltpu.sync_copy(lg_v, out_hbm.at[pl.ds(o0 + bi * TB, TB)])
        return carry

    lax.fori_loop(0, TPW // TB, block, 0)


def _sc_project(x, W, b):
    mesh = plsc.VectorSubcoreMesh(core_axis_name="c", subcore_axis_name="s")
    k = pl.kernel(
        _scproj_body,
        out_type=jax.ShapeDtypeStruct((SC_TOKENS, NCH), jnp.float32),
        mesh=mesh,
        compiler_params=pltpu.CompilerParams(use_tc_tiling_on_sc=False),
        scratch_types=[
            pltpu.VMEM((DM, NCH), jnp.float32),
            pltpu.VMEM((NCH,), jnp.float32),
            pltpu.VMEM((TB, DM), jnp.float32),
            pltpu.VMEM((TB, NCH), jnp.float32),
        ],
    )
    return k(x, W, b)


def _sc_body(lg_hbm, actors_hbm, out_lg, idx_v, rows_v, sem):
    ncores = lax.axis_size("c")
    wid = lax.axis_index("s") * ncores + lax.axis_index("c")
    base = wid * BPW
    pltpu.sync_copy(actors_hbm.at[pl.ds(base, BPW)], idx_v)
    pltpu.async_copy(lg_hbm.at[idx_v], rows_v, sem).wait()
    pltpu.sync_copy(rows_v, out_lg.at[pl.ds(base, BPW)])


def _sc_gather(lg_all, actors):
    mesh = plsc.VectorSubcoreMesh(core_axis_name="c", subcore_axis_name="s")
    k = pl.kernel(
        _sc_body,
        out_type=jax.ShapeDtypeStruct((ACTORS, NCH), jnp.float32),
        mesh=mesh,
        compiler_params=pltpu.CompilerParams(use_tc_tiling_on_sc=False),
        scratch_types=[
            pltpu.VMEM((BPW,), jnp.int32),
            pltpu.VMEM((BPW, NCH), jnp.float32),
            pltpu.SemaphoreType.DMA,
        ],
    )
    return k(lg_all, actors)


def _epi_body(lg_ref, pa_ref, lp_ref, logp_ref, ent_ref):
    logits = lg_ref[...]
    m = jnp.max(logits, axis=-1, keepdims=True)
    e = jnp.exp(logits - m)
    lse = m + jnp.log(jnp.sum(e, axis=-1, keepdims=True))
    lp = logits - lse
    lp_ref[...] = lp
    cols = lax.broadcasted_iota(jnp.int32, (ROW_BLK, NCH), 1)
    sel = cols == pa_ref[...]
    logp_ref[...] = jnp.sum(jnp.where(sel, lp, 0.0), axis=-1, keepdims=True)
    ent_ref[...] = -jnp.sum(jnp.exp(lp) * lp, axis=-1, keepdims=True)


def _epilogue(logits_g, pa2):
    return pl.pallas_call(
        _epi_body,
        grid=(ACTORS // ROW_BLK,),
        in_specs=[
            pl.BlockSpec((ROW_BLK, NCH), lambda i: (i, 0)),
            pl.BlockSpec((ROW_BLK, 1), lambda i: (i, 0)),
        ],
        out_specs=[
            pl.BlockSpec((ROW_BLK, NCH), lambda i: (i, 0)),
            pl.BlockSpec((ROW_BLK, 1), lambda i: (i, 0)),
            pl.BlockSpec((ROW_BLK, 1), lambda i: (i, 0)),
        ],
        out_shape=[
            jax.ShapeDtypeStruct((ACTORS, NCH), jnp.float32),
            jax.ShapeDtypeStruct((ACTORS, 1), jnp.float32),
            jax.ShapeDtypeStruct((ACTORS, 1), jnp.float32),
        ],
    )(logits_g, pa2)


def kernel(x, actors, lengths, prev_actions, W, b):
    lg_sc = _sc_project(x, W, b)
    lg_tc = _project(x, W, b.reshape(1, NCH))
    lg_all = jnp.concatenate([lg_tc, lg_sc], axis=0)
    lg_g = _sc_gather(lg_all, actors)
    log_probs, logprob, entropy = _epilogue(
        lg_g, prev_actions.reshape(ACTORS, 1))
    return (prev_actions, lengths, logprob.reshape(ACTORS),
            entropy.reshape(ACTORS), log_probs)


# speedup vs baseline: 1.4557x; 1.4557x over previous
import jax
import jax.numpy as jnp
from jax import lax
from jax.experimental import pallas as pl
from jax.experimental.pallas import tpu as pltpu
from jax.experimental.pallas import tpu_sc as plsc

TOKENS = 32768
ACTORS = 32768
DM = 2048
NCH = 16

ROW_BLK = 2048
NWORKERS = 32
BPW = ACTORS // NWORKERS

SC_TOKENS = 2048
TC_TOKENS = TOKENS - SC_TOKENS
TPW = SC_TOKENS // NWORKERS
TB = 8
KC = 16
KN = DM // KC


def _proj_body(x_ref, w_ref, b_ref, lg_ref):
    logits = jnp.dot(x_ref[...], w_ref[...], preferred_element_type=jnp.float32)
    lg_ref[...] = logits + b_ref[...]


def _project(x, W, b2):
    return pl.pallas_call(
        _proj_body,
        grid=(TC_TOKENS // ROW_BLK,),
        in_specs=[
            pl.BlockSpec((ROW_BLK, DM), lambda i: (i, 0)),
            pl.BlockSpec((DM, NCH), lambda i: (0, 0)),
            pl.BlockSpec((1, NCH), lambda i: (0, 0)),
        ],
        out_specs=pl.BlockSpec((ROW_BLK, NCH), lambda i: (i, 0)),
        out_shape=jax.ShapeDtypeStruct((TC_TOKENS, NCH), jnp.float32),
        compiler_params=pltpu.CompilerParams(
            vmem_limit_bytes=100 * 1024 * 1024),
    )(x, W, b2)


def _scproj_body(x_hbm, w_hbm, b_hbm, out_hbm, w_v, b_v, xb_v, lg_v):
    ncores = lax.axis_size("c")
    wid = lax.axis_index("s") * ncores + lax.axis_index("c")
    t0 = TC_TOKENS + wid * TPW
    o0 = wid * TPW
    pltpu.sync_copy(w_hbm, w_v)
    pltpu.sync_copy(b_hbm, b_v)
    bvec = b_v[...]

    def block(bi, carry):
        pltpu.sync_copy(x_hbm.at[pl.ds(t0 + bi * TB, TB)], xb_v)

        def kchunk(kc, accs):
            k0 = kc * KC
            wvs = [w_v[k0 + kk, :] for kk in range(KC)]
            new = list(accs)
            for t in range(TB):
                xv = xb_v[t, pl.ds(k0, KC)]
                for kk in range(KC):
                    new[t] = new[t] + xv[kk] * wvs[kk]
            return tuple(new)

        accs = lax.fori_loop(0, KN, kchunk, (bvec,) * TB)
        for t in range(TB):
            lg_v[t, :] = accs[t]
        pltpu.sync_copy(lg_v, out_hbm.at[pl.ds(o0 + bi * TB, TB)])
        return carry

    lax.fori_loop(0, TPW // TB, block, 0)


def _sc_project(x, W, b):
    mesh = plsc.VectorSubcoreMesh(core_axis_name="c", subcore_axis_name="s")
    k = pl.kernel(
        _scproj_body,
        out_type=jax.ShapeDtypeStruct((SC_TOKENS, NCH), jnp.float32),
        mesh=mesh,
        compiler_params=pltpu.CompilerParams(use_tc_tiling_on_sc=False),
        scratch_types=[
            pltpu.VMEM((DM, NCH), jnp.float32),
            pltpu.VMEM((NCH,), jnp.float32),
            pltpu.VMEM((TB, DM), jnp.float32),
            pltpu.VMEM((TB, NCH), jnp.float32),
        ],
    )
    return k(x, W, b)


def _sc_body(lg_hbm, actors_hbm, out_lg, idx_v, rows_v, sem):
    ncores = lax.axis_size("c")
    wid = lax.axis_index("s") * ncores + lax.axis_index("c")
    base = wid * BPW
    pltpu.sync_copy(actors_hbm.at[pl.ds(base, BPW)], idx_v)
    pltpu.async_copy(lg_hbm.at[idx_v], rows_v, sem).wait()
    pltpu.sync_copy(rows_v, out_lg.at[pl.ds(base, BPW)])


def _sc_gather(lg_all, actors):
    mesh = plsc.VectorSubcoreMesh(core_axis_name="c", subcore_axis_name="s")
    k = pl.kernel(
        _sc_body,
        out_type=jax.ShapeDtypeStruct((ACTORS, NCH), jnp.float32),
        mesh=mesh,
        compiler_params=pltpu.CompilerParams(use_tc_tiling_on_sc=False),
        scratch_types=[
            pltpu.VMEM((BPW,), jnp.int32),
            pltpu.VMEM((BPW, NCH), jnp.float32),
            pltpu.SemaphoreType.DMA,
        ],
    )
    return k(lg_all, actors)


def _epi_body(lg_ref, pa_ref, lp_ref, logp_ref, ent_ref):
    logits = lg_ref[...]
    m = jnp.max(logits, axis=-1, keepdims=True)
    e = jnp.exp(logits - m)
    lse = m + jnp.log(jnp.sum(e, axis=-1, keepdims=True))
    lp = logits - lse
    lp_ref[...] = lp
    cols = lax.broadcasted_iota(jnp.int32, (ROW_BLK, NCH), 1)
    sel = cols == pa_ref[...]
    logp_ref[...] = jnp.sum(jnp.where(sel, lp, 0.0), axis=-1, keepdims=True)
    ent_ref[...] = -jnp.sum(jnp.exp(lp) * lp, axis=-1, keepdims=True)


def _epilogue(logits_g, pa2):
    return pl.pallas_call(
        _epi_body,
        grid=(ACTORS // ROW_BLK,),
        in_specs=[
            pl.BlockSpec((ROW_BLK, NCH), lambda i: (i, 0)),
            pl.BlockSpec((ROW_BLK, 1), lambda i: (i, 0)),
        ],
        out_specs=[
            pl.BlockSpec((ROW_BLK, NCH), lambda i: (i, 0)),
            pl.BlockSpec((ROW_BLK, 1), lambda i: (i, 0)),
            pl.BlockSpec((ROW_BLK, 1), lambda i: (i, 0)),
        ],
        out_shape=[
            jax.ShapeDtypeStruct((ACTORS, NCH), jnp.float32),
            jax.ShapeDtypeStruct((ACTORS, 1), jnp.float32),
            jax.ShapeDtypeStruct((ACTORS, 1), jnp.float32),
        ],
    )(logits_g, pa2)


def kernel(x, actors, lengths, prev_actions, W, b):
    lg_sc = _sc_project(x, W, b)
    lg_tc = _project(x, W, b.reshape(1, NCH))
    lg_all = jnp.concatenate([lg_tc, lg_sc], axis=0)
    lg_g = _sc_gather(lg_all, actors)
    log_probs, logprob, entropy = _epilogue(
        lg_g, prev_actions.reshape(ACTORS, 1))
    return (prev_actions, lengths, logprob.reshape(ACTORS),
            entropy.reshape(ACTORS), log_probs)


# speedup vs baseline: 4.3503x; 2.9884x over previous
import jax
import jax.numpy as jnp
from jax import lax
from jax.experimental import pallas as pl
from jax.experimental.pallas import tpu as pltpu
from jax.experimental.pallas import tpu_sc as plsc

TOKENS = 32768
ACTORS = 32768
DM = 2048
NCH = 16

ROW_BLK = 2048
NWORKERS = 32
BPW = ACTORS // NWORKERS


def _proj_body(x_ref, w_ref, b_ref, lp_ref):
    logits = jnp.dot(x_ref[...], w_ref[...], preferred_element_type=jnp.float32)
    logits = logits + b_ref[...]
    m = jnp.max(logits, axis=-1, keepdims=True)
    e = jnp.exp(logits - m)
    lse = m + jnp.log(jnp.sum(e, axis=-1, keepdims=True))
    lp_ref[...] = logits - lse


def _project(x, W, b2):
    return pl.pallas_call(
        _proj_body,
        grid=(TOKENS // ROW_BLK,),
        in_specs=[
            pl.BlockSpec((ROW_BLK, DM), lambda i: (i, 0)),
            pl.BlockSpec((DM, NCH), lambda i: (0, 0)),
            pl.BlockSpec((1, NCH), lambda i: (0, 0)),
        ],
        out_specs=pl.BlockSpec((ROW_BLK, NCH), lambda i: (i, 0)),
        out_shape=jax.ShapeDtypeStruct((TOKENS, NCH), jnp.float32),
        compiler_params=pltpu.CompilerParams(
            dimension_semantics=("parallel",),
            vmem_limit_bytes=100 * 1024 * 1024),
    )(x, W, b2)


def _sc_body(lp_hbm, actors_hbm, out_lp, idx_v, rows_v, sem):
    ncores = lax.axis_size("c")
    wid = lax.axis_index("s") * ncores + lax.axis_index("c")
    base = wid * BPW
    pltpu.sync_copy(actors_hbm.at[pl.ds(base, BPW)], idx_v)
    pltpu.async_copy(lp_hbm.at[idx_v], rows_v, sem).wait()
    pltpu.sync_copy(rows_v, out_lp.at[pl.ds(base, BPW)])


def _sc_gather(lp_all, actors):
    mesh = plsc.VectorSubcoreMesh(core_axis_name="c", subcore_axis_name="s")
    k = pl.kernel(
        _sc_body,
        out_type=jax.ShapeDtypeStruct((ACTORS, NCH), jnp.float32),
        mesh=mesh,
        compiler_params=pltpu.CompilerParams(use_tc_tiling_on_sc=False),
        scratch_types=[
            pltpu.VMEM((BPW,), jnp.int32),
            pltpu.VMEM((BPW, NCH), jnp.float32),
            pltpu.SemaphoreType.DMA,
        ],
    )
    return k(lp_all, actors)


def _epi_body(lp_g_ref, pa_ref, logp_ref, ent_ref):
    lp = lp_g_ref[...]
    cols = lax.broadcasted_iota(jnp.int32, (ROW_BLK, NCH), 1)
    sel = cols == pa_ref[...]
    logp_ref[...] = jnp.sum(jnp.where(sel, lp, 0.0), axis=-1, keepdims=True)
    ent_ref[...] = -jnp.sum(jnp.exp(lp) * lp, axis=-1, keepdims=True)


def _epilogue(log_probs, pa2):
    return pl.pallas_call(
        _epi_body,
        grid=(ACTORS // ROW_BLK,),
        in_specs=[
            pl.BlockSpec((ROW_BLK, NCH), lambda i: (i, 0)),
            pl.BlockSpec((ROW_BLK, 1), lambda i: (i, 0)),
        ],
        out_specs=[
            pl.BlockSpec((ROW_BLK, 1), lambda i: (i, 0)),
            pl.BlockSpec((ROW_BLK, 1), lambda i: (i, 0)),
        ],
        out_shape=[
            jax.ShapeDtypeStruct((ACTORS, 1), jnp.float32),
            jax.ShapeDtypeStruct((ACTORS, 1), jnp.float32),
        ],
    )(log_probs, pa2)


def kernel(x, actors, lengths, prev_actions, W, b):
    lp_all = _project(x, W, b.reshape(1, NCH))
    log_probs = _sc_gather(lp_all, actors)
    logprob, entropy = _epilogue(log_probs, prev_actions.reshape(ACTORS, 1))
    return (prev_actions, lengths, logprob.reshape(ACTORS),
            entropy.reshape(ACTORS), log_probs)


# speedup vs baseline: 4.7034x; 1.0812x over previous
import jax
import jax.numpy as jnp
from jax import lax
from jax.experimental import pallas as pl
from jax.experimental.pallas import tpu as pltpu
from jax.experimental.pallas import tpu_sc as plsc

TOKENS = 32768
ACTORS = 32768
DM = 2048
NCH = 16

ROW_BLK = 2048
NWORKERS = 32
BPW = ACTORS // NWORKERS


def _proj_body(x_ref, w_ref, b_ref, lp_ref):
    logits = jnp.dot(x_ref[...], w_ref[...], preferred_element_type=jnp.float32)
    logits = logits + b_ref[...]
    m = jnp.max(logits, axis=-1, keepdims=True)
    e = jnp.exp(logits - m)
    lse = m + jnp.log(jnp.sum(e, axis=-1, keepdims=True))
    lp_ref[...] = logits - lse


def _project(x, W, b2):
    return pl.pallas_call(
        _proj_body,
        grid=(TOKENS // ROW_BLK,),
        in_specs=[
            pl.BlockSpec((ROW_BLK, DM), lambda i: (i, 0)),
            pl.BlockSpec((DM, NCH), lambda i: (0, 0)),
            pl.BlockSpec((1, NCH), lambda i: (0, 0)),
        ],
        out_specs=pl.BlockSpec((ROW_BLK, NCH), lambda i: (i, 0)),
        out_shape=jax.ShapeDtypeStruct((TOKENS, NCH), jnp.float32),
        compiler_params=pltpu.CompilerParams(
            dimension_semantics=("parallel",),
            vmem_limit_bytes=100 * 1024 * 1024),
    )(x, W, b2)


def _sc_body(lp_hbm, actors_hbm, out_lp, idx_v, rows_v, sem):
    ncores = lax.axis_size("c")
    wid = lax.axis_index("s") * ncores + lax.axis_index("c")
    base = wid * BPW
    pltpu.sync_copy(actors_hbm.at[pl.ds(base, BPW)], idx_v)
    pltpu.async_copy(lp_hbm.at[idx_v], rows_v, sem).wait()
    pltpu.sync_copy(rows_v, out_lp.at[pl.ds(base, BPW)])


def _sc_gather(lp_all, actors):
    mesh = plsc.VectorSubcoreMesh(core_axis_name="c", subcore_axis_name="s")
    k = pl.kernel(
        _sc_body,
        out_type=jax.ShapeDtypeStruct((ACTORS, NCH), jnp.float32),
        mesh=mesh,
        compiler_params=pltpu.CompilerParams(use_tc_tiling_on_sc=False),
        scratch_types=[
            pltpu.VMEM((BPW,), jnp.int32),
            pltpu.VMEM((BPW, NCH), jnp.float32),
            pltpu.SemaphoreType.DMA,
        ],
    )
    return k(lp_all, actors)


def _epi_body(lp_g_ref, pa_ref, logp_ref, ent_ref):
    lp = lp_g_ref[...]
    cols = lax.broadcasted_iota(jnp.int32, (ROW_BLK, NCH), 1)
    sel = cols == pa_ref[...][:, None]
    logp_ref[...] = jnp.sum(jnp.where(sel, lp, 0.0), axis=-1)
    ent_ref[...] = -jnp.sum(jnp.exp(lp) * lp, axis=-1)


def _epilogue(log_probs, prev_actions):
    return pl.pallas_call(
        _epi_body,
        grid=(ACTORS // ROW_BLK,),
        in_specs=[
            pl.BlockSpec((ROW_BLK, NCH), lambda i: (i, 0)),
            pl.BlockSpec((ROW_BLK,), lambda i: (i,)),
        ],
        out_specs=[
            pl.BlockSpec((ROW_BLK,), lambda i: (i,)),
            pl.BlockSpec((ROW_BLK,), lambda i: (i,)),
        ],
        out_shape=[
            jax.ShapeDtypeStruct((ACTORS,), jnp.float32),
            jax.ShapeDtypeStruct((ACTORS,), jnp.float32),
        ],
    )(log_probs, prev_actions)


def kernel(x, actors, lengths, prev_actions, W, b):
    lp_all = _project(x, W, b.reshape(1, NCH))
    log_probs = _sc_gather(lp_all, actors)
    logprob, entropy = _epilogue(log_probs, prev_actions)
    return (prev_actions, lengths, logprob, entropy, log_probs)


# speedup vs baseline: 4.7335x; 1.0064x over previous
import jax
import jax.numpy as jnp
from jax import lax
from jax.experimental import pallas as pl
from jax.experimental.pallas import tpu as pltpu
from jax.experimental.pallas import tpu_sc as plsc

TOKENS = 32768
ACTORS = 32768
DM = 2048
NCH = 16

ROW_BLK = 2048
EPI_BLK = 8192
NWORKERS = 32
BPW = ACTORS // NWORKERS


def _proj_body(x_ref, w_ref, b_ref, lp_ref):
    logits = jnp.dot(x_ref[...], w_ref[...], preferred_element_type=jnp.float32)
    logits = logits + b_ref[...]
    m = jnp.max(logits, axis=-1, keepdims=True)
    e = jnp.exp(logits - m)
    lse = m + jnp.log(jnp.sum(e, axis=-1, keepdims=True))
    lp_ref[...] = logits - lse


def _project(x, W, b2):
    return pl.pallas_call(
        _proj_body,
        grid=(TOKENS // ROW_BLK,),
        in_specs=[
            pl.BlockSpec((ROW_BLK, DM), lambda i: (i, 0)),
            pl.BlockSpec((DM, NCH), lambda i: (0, 0)),
            pl.BlockSpec((1, NCH), lambda i: (0, 0)),
        ],
        out_specs=pl.BlockSpec((ROW_BLK, NCH), lambda i: (i, 0)),
        out_shape=jax.ShapeDtypeStruct((TOKENS, NCH), jnp.float32),
        compiler_params=pltpu.CompilerParams(
            dimension_semantics=("parallel",),
            vmem_limit_bytes=100 * 1024 * 1024),
    )(x, W, b2)


def _sc_body(lp_hbm, actors_hbm, out_lp, idx_v, rows_v, sem):
    ncores = lax.axis_size("c")
    wid = lax.axis_index("s") * ncores + lax.axis_index("c")
    base = wid * BPW
    pltpu.sync_copy(actors_hbm.at[pl.ds(base, BPW)], idx_v)
    pltpu.async_copy(lp_hbm.at[idx_v], rows_v, sem).wait()
    pltpu.sync_copy(rows_v, out_lp.at[pl.ds(base, BPW)])


def _sc_gather(lp_all, actors):
    mesh = plsc.VectorSubcoreMesh(core_axis_name="c", subcore_axis_name="s")
    k = pl.kernel(
        _sc_body,
        out_type=jax.ShapeDtypeStruct((ACTORS, NCH), jnp.float32),
        mesh=mesh,
        compiler_params=pltpu.CompilerParams(use_tc_tiling_on_sc=False),
        scratch_types=[
            pltpu.VMEM((BPW,), jnp.int32),
            pltpu.VMEM((BPW, NCH), jnp.float32),
            pltpu.SemaphoreType.DMA,
        ],
    )
    return k(lp_all, actors)


def _epi_body(lp_g_ref, pa_ref, logp_ref, ent_ref):
    lp = lp_g_ref[...]
    cols = lax.broadcasted_iota(jnp.int32, (EPI_BLK, NCH), 1)
    sel = cols == pa_ref[...][:, None]
    logp_ref[...] = jnp.sum(jnp.where(sel, lp, 0.0), axis=-1)
    ent_ref[...] = -jnp.sum(jnp.exp(lp) * lp, axis=-1)


def _epilogue(log_probs, prev_actions):
    return pl.pallas_call(
        _epi_body,
        grid=(ACTORS // EPI_BLK,),
        in_specs=[
            pl.BlockSpec((EPI_BLK, NCH), lambda i: (i, 0)),
            pl.BlockSpec((EPI_BLK,), lambda i: (i,)),
        ],
        out_specs=[
            pl.BlockSpec((EPI_BLK,), lambda i: (i,)),
            pl.BlockSpec((EPI_BLK,), lambda i: (i,)),
        ],
        out_shape=[
            jax.ShapeDtypeStruct((ACTORS,), jnp.float32),
            jax.ShapeDtypeStruct((ACTORS,), jnp.float32),
        ],
    )(log_probs, prev_actions)


def kernel(x, actors, lengths, prev_actions, W, b):
    lp_all = _project(x, W, b.reshape(1, NCH))
    log_probs = _sc_gather(lp_all, actors)
    logprob, entropy = _epilogue(log_probs, prev_actions)
    return (prev_actions, lengths, logprob, entropy, log_probs)
